# baseline (speedup 1.0000x reference)
import functools

import jax
import jax.numpy as jnp
from jax import lax
from jax.experimental import pallas as pl
from jax.experimental.pallas import tpu as pltpu
from jax.experimental.pallas import tpu_sc as plsc

B, S, D, H, E, K, FF = 1, 2048, 1024, 16, 8, 2, 4096
HD = D // H
EPS = 1e-05
CAP = 1024
T = B * S

TILE = 128
PAD_T = T * K + E * TILE
NUM_TILES = PAD_T // TILE

_NC, _NS = 2, 16
_NW = _NC * _NS


def _norm_matmul_body(x_ref, nw_ref, w_ref, o_ref):
    x = x_ref[...]
    h = x * lax.rsqrt(jnp.mean(x * x, axis=-1, keepdims=True) + EPS) * nw_ref[...]
    o_ref[...] = jnp.dot(h, w_ref[...], preferred_element_type=jnp.float32)


def _norm_matmul(x, nw, w, block_rows=256):
    n = w.shape[1]
    return pl.pallas_call(
        _norm_matmul_body,
        grid=(T // block_rows,),
        in_specs=[
            pl.BlockSpec((block_rows, D), lambda i: (i, 0)),
            pl.BlockSpec((1, D), lambda i: (0, 0)),
            pl.BlockSpec((D, n), lambda i: (0, 0)),
        ],
        out_specs=pl.BlockSpec((block_rows, n), lambda i: (i, 0)),
        out_shape=jax.ShapeDtypeStruct((T, n), jnp.float32),
        compiler_params=pltpu.CompilerParams(
            dimension_semantics=("parallel",)),
    )(x, nw, w)


def _attn_body(q_ref, k_ref, v_ref, o_ref):
    i = pl.program_id(1)
    bq = q_ref.shape[1]
    bk = 256
    q = q_ref[0]

    def step(j, carry):
        acc, m, l = carry
        k = k_ref[0, pl.ds(j * bk, bk), :]
        v = v_ref[0, pl.ds(j * bk, bk), :]
        s = lax.dot_general(q, k, (((1,), (1,)), ((), ())),
                            preferred_element_type=jnp.float32) * 0.125
        rows = i * bq + lax.broadcasted_iota(jnp.int32, (bq, bk), 0)
        cols = j * bk + lax.broadcasted_iota(jnp.int32, (bq, bk), 1)
        s = jnp.where(rows >= cols, s, jnp.float32(-1e9))
        m2 = jnp.maximum(m, jnp.max(s, axis=-1, keepdims=True))
        p = jnp.exp(s - m2)
        corr = jnp.exp(m - m2)
        l2 = l * corr + jnp.sum(p, axis=-1, keepdims=True)
        acc2 = acc * corr + jnp.dot(p, v, preferred_element_type=jnp.float32)
        return acc2, m2, l2

    acc0 = jnp.zeros((bq, HD), jnp.float32)
    m0 = jnp.full((bq, 1), -1e30, jnp.float32)
    l0 = jnp.zeros((bq, 1), jnp.float32)
    n_chunks = (i + 1) * bq // bk
    acc, m, l = lax.fori_loop(0, n_chunks, step, (acc0, m0, l0))
    o_ref[0] = acc / l


def _attention(q, k, v, block_q=256):
    return pl.pallas_call(
        _attn_body,
        grid=(H, S // block_q),
        in_specs=[
            pl.BlockSpec((1, block_q, HD), lambda h, i: (h, i, 0)),
            pl.BlockSpec((1, S, HD), lambda h, i: (h, 0, 0)),
            pl.BlockSpec((1, S, HD), lambda h, i: (h, 0, 0)),
        ],
        out_specs=pl.BlockSpec((1, block_q, HD), lambda h, i: (h, i, 0)),
        out_shape=jax.ShapeDtypeStruct((H, S, HD), jnp.float32),
        compiler_params=pltpu.CompilerParams(
            dimension_semantics=("parallel", "parallel")),
    )(q, k, v)


def _proj_res_body(o_ref, w_ref, x_ref, y_ref):
    y_ref[...] = x_ref[...] + jnp.dot(o_ref[...], w_ref[...],
                                      preferred_element_type=jnp.float32)


def _proj_residual(o, w, x, block_rows=256):
    return pl.pallas_call(
        _proj_res_body,
        grid=(T // block_rows,),
        in_specs=[
            pl.BlockSpec((block_rows, D), lambda i: (i, 0)),
            pl.BlockSpec((D, D), lambda i: (0, 0)),
            pl.BlockSpec((block_rows, D), lambda i: (i, 0)),
        ],
        out_specs=pl.BlockSpec((block_rows, D), lambda i: (i, 0)),
        out_shape=jax.ShapeDtypeStruct((T, D), jnp.float32),
        compiler_params=pltpu.CompilerParams(
            dimension_semantics=("parallel",)),
    )(o, w, x)


def _norm2_body(x_ref, nw_ref, rw_ref, f_ref, l_ref):
    x = x_ref[...]
    h = x * lax.rsqrt(jnp.mean(x * x, axis=-1, keepdims=True) + EPS) * nw_ref[...]
    f_ref[...] = h.astype(jnp.bfloat16)
    l_ref[...] = jnp.dot(h, rw_ref[...], preferred_element_type=jnp.float32)


def _norm_and_router(x1, nw, rw_pad, block_rows=256):
    return pl.pallas_call(
        _norm2_body,
        grid=(T // block_rows,),
        in_specs=[
            pl.BlockSpec((block_rows, D), lambda i: (i, 0)),
            pl.BlockSpec((1, D), lambda i: (0, 0)),
            pl.BlockSpec((D, 128), lambda i: (0, 0)),
        ],
        out_specs=[
            pl.BlockSpec((block_rows, D), lambda i: (i, 0)),
            pl.BlockSpec((block_rows, 128), lambda i: (i, 0)),
        ],
        out_shape=[
            jax.ShapeDtypeStruct((T, D), jnp.bfloat16),
            jax.ShapeDtypeStruct((T, 128), jnp.float32),
        ],
        compiler_params=pltpu.CompilerParams(
            dimension_semantics=("parallel",)),
    )(x1, nw, rw_pad)


def _ffn_body(te_ref, xg_ref, w1_ref, b1_ref, w2_ref, b2_ref, o_ref):
    x = xg_ref[...]
    h = jnp.dot(x, w1_ref[0], preferred_element_type=jnp.float32) + b1_ref[0]
    h = jax.nn.gelu(h)
    y = jnp.dot(h.astype(jnp.bfloat16), w2_ref[0],
                preferred_element_type=jnp.float32) + b2_ref[0]
    o_ref[...] = y.astype(jnp.bfloat16)


def _moe_ffn(xg, tile_expert, w1, b1, w2, b2):
    grid_spec = pltpu.PrefetchScalarGridSpec(
        num_scalar_prefetch=1,
        grid=(NUM_TILES,),
        in_specs=[
            pl.BlockSpec((TILE, D), lambda t, te: (t, 0)),
            pl.BlockSpec((1, D, FF), lambda t, te: (te[t], 0, 0)),
            pl.BlockSpec((1, 1, FF), lambda t, te: (te[t], 0, 0)),
            pl.BlockSpec((1, FF, D), lambda t, te: (te[t], 0, 0)),
            pl.BlockSpec((1, 1, D), lambda t, te: (te[t], 0, 0)),
        ],
        out_specs=pl.BlockSpec((TILE, D), lambda t, te: (t, 0)),
    )
    return pl.pallas_call(
        _ffn_body,
        grid_spec=grid_spec,
        out_shape=jax.ShapeDtypeStruct((PAD_T, D), jnp.bfloat16),
    )(tile_expert, xg, w1, b1, w2, b2)


def _combine_body(x1_ref, a0_ref, a1_ref, s0_ref, s1_ref, y_ref):
    y_ref[...] = (x1_ref[...] + a0_ref[...].astype(jnp.float32) * s0_ref[...]
                  + a1_ref[...].astype(jnp.float32) * s1_ref[...])


def _combine(x1, a0, a1, s0, s1, block_rows=512):
    return pl.pallas_call(
        _combine_body,
        grid=(T // block_rows,),
        in_specs=[
            pl.BlockSpec((block_rows, D), lambda i: (i, 0)),
            pl.BlockSpec((block_rows, D), lambda i: (i, 0)),
            pl.BlockSpec((block_rows, D), lambda i: (i, 0)),
            pl.BlockSpec((block_rows, 1), lambda i: (i, 0)),
            pl.BlockSpec((block_rows, 1), lambda i: (i, 0)),
        ],
        out_specs=pl.BlockSpec((block_rows, D), lambda i: (i, 0)),
        out_shape=jax.ShapeDtypeStruct((T, D), jnp.float32),
        compiler_params=pltpu.CompilerParams(
            dimension_semantics=("parallel",)),
    )(x1, a0, a1, s0, s1)


def _sc_gather_rows(table, idx, n_rows):
    v_rows = table.shape[0]
    w32 = D // 2
    t3 = jax.lax.bitcast_convert_type(table.reshape(v_rows, w32, 2), jnp.int32)
    b_per_w = n_rows // _NW
    ch = b_per_w
    n_chunks = 1
    while ch > 128:
        ch //= 2
        n_chunks *= 2
    mesh = plsc.VectorSubcoreMesh(core_axis_name="c", subcore_axis_name="s",
                                  num_cores=_NC, num_subcores=_NS)

    @functools.partial(
        pl.kernel, mesh=mesh,
        out_type=jax.ShapeDtypeStruct((n_rows, w32), jnp.int32),
        scratch_types=[
            pltpu.VMEM((ch,), jnp.int32),
            pltpu.VMEM((ch, w32), jnp.int32),
            pltpu.SemaphoreType.DMA,
        ],
    )
    def k(table_hbm, idx_hbm, out_hbm, idx_v, rows_v, sem):
        wid = lax.axis_index("s") * _NC + lax.axis_index("c")
        base = wid * b_per_w
        for c in range(n_chunks):
            off = base + c * ch
            pltpu.sync_copy(idx_hbm.at[pl.ds(off, ch)], idx_v)
            pltpu.async_copy(table_hbm.at[idx_v], rows_v, sem).wait()
            pltpu.sync_copy(rows_v, out_hbm.at[pl.ds(off, ch)])

    g32 = k(t3, idx)
    return jax.lax.bitcast_convert_type(g32, jnp.bfloat16).reshape(n_rows, D)


def kernel(x, attn_norm_w, Wq, Wk, Wv, Wo, mlp_norm_w, router_w, router_b,
           ew1, eb1, ew2, eb2):
    x2d = x.reshape(T, D)
    nw1 = attn_norm_w.reshape(1, D)
    nw2 = mlp_norm_w.reshape(1, D)

    wqkv = jnp.concatenate([Wq, Wk, Wv], axis=1)
    qkv = _norm_matmul(x2d, nw1, wqkv)
    qkv3 = qkv.reshape(S, 3, H, HD).transpose(1, 2, 0, 3)
    o = _attention(qkv3[0], qkv3[1], qkv3[2])
    o2d = o.transpose(1, 0, 2).reshape(T, D)
    x1 = _proj_residual(o2d, Wo, x2d)

    rw_pad = jnp.zeros((D, 128), jnp.float32).at[:, :E].set(router_w)
    flat, logits_pad = _norm_and_router(x1, nw2, rw_pad)
    logits = logits_pad[:, :E] + router_b
    probs = jax.nn.softmax(logits, axis=-1)
    topk_scores, topk_idx = lax.top_k(probs, K)

    mean_probs = probs.mean(axis=0)
    flat_sel = topk_idx.reshape(-1)
    oh = (flat_sel[:, None] == jnp.arange(E)[None, :]).astype(jnp.int32)
    counts = oh.sum(axis=0)
    expert_counts = counts.astype(jnp.int32)
    fraction = counts.astype(jnp.float32) / jnp.float32(T * K)
    balancing_loss = jnp.float32(E) * jnp.sum(mean_probs * fraction)

    rank_r = (jnp.cumsum(oh, axis=0) * oh).sum(axis=1) - 1
    seg_pad = ((counts + TILE - 1) // TILE) * TILE
    po = jnp.concatenate([jnp.zeros((1,), jnp.int32),
                          jnp.cumsum(seg_pad)]).astype(jnp.int32)
    p_of_r = po[flat_sel] + rank_r
    valid_r = (rank_r < CAP).astype(jnp.float32)
    sorted_src = jnp.zeros((PAD_T,), jnp.int32).at[p_of_r].set(
        jnp.arange(T * K, dtype=jnp.int32))
    gather_idx = sorted_src // K
    tile_expert = jnp.minimum(
        jnp.searchsorted(po[1:], jnp.arange(NUM_TILES, dtype=jnp.int32) * TILE,
                         side="right"), E - 1).astype(jnp.int32)

    w_scr = jnp.transpose(topk_scores).reshape(-1)
    wv = w_scr * valid_r
    s0 = wv[:T].reshape(T, 1)
    s1 = wv[T:].reshape(T, 1)

    xg = _sc_gather_rows(flat, gather_idx, PAD_T)

    out_sorted = _moe_ffn(xg, tile_expert,
                          ew1.astype(jnp.bfloat16), eb1.reshape(E, 1, FF),
                          ew2.astype(jnp.bfloat16), eb2.reshape(E, 1, D))

    ab = _sc_gather_rows(out_sorted, p_of_r.astype(jnp.int32), 2 * T)
    a0, a1 = ab[:T], ab[T:]

    x2 = _combine(x1, a0, a1, s0, s1)
    return (x2.reshape(B, S, D), balancing_loss, expert_counts)


# speedup vs baseline: 1.5421x; 1.5421x over previous
import functools

import jax
import jax.numpy as jnp
from jax import lax
from jax.experimental import pallas as pl
from jax.experimental.pallas import tpu as pltpu
from jax.experimental.pallas import tpu_sc as plsc

B, S, D, H, E, K, FF = 1, 2048, 1024, 16, 8, 2, 4096
HD = D // H
EPS = 1e-05
CAP = 1024
T = B * S

TILE = 128
PAD_T = T * K + E * TILE
NUM_TILES = PAD_T // TILE

_NC, _NS = 2, 16
_NW = _NC * _NS


def _norm_matmul_body(x_ref, nw_ref, w_ref, o_ref):
    x = x_ref[...]
    h = x * lax.rsqrt(jnp.mean(x * x, axis=-1, keepdims=True) + EPS) * nw_ref[...]
    o_ref[...] = jnp.dot(h, w_ref[...], preferred_element_type=jnp.float32)


def _norm_matmul(x, nw, w, block_rows=256):
    n = w.shape[1]
    return pl.pallas_call(
        _norm_matmul_body,
        grid=(T // block_rows,),
        in_specs=[
            pl.BlockSpec((block_rows, D), lambda i: (i, 0)),
            pl.BlockSpec((1, D), lambda i: (0, 0)),
            pl.BlockSpec((D, n), lambda i: (0, 0)),
        ],
        out_specs=pl.BlockSpec((block_rows, n), lambda i: (i, 0)),
        out_shape=jax.ShapeDtypeStruct((T, n), jnp.float32),
        compiler_params=pltpu.CompilerParams(
            dimension_semantics=("parallel",)),
    )(x, nw, w)


def _attn_body(q_ref, k_ref, v_ref, o_ref):
    i = pl.program_id(1)
    bq = q_ref.shape[1]
    q = q_ref[0]
    k = k_ref[0]
    s = lax.dot_general(q, k, (((1,), (1,)), ((), ())),
                        preferred_element_type=jnp.float32) * 0.125
    rows = i * bq + lax.broadcasted_iota(jnp.int32, (bq, S), 0)
    cols = lax.broadcasted_iota(jnp.int32, (bq, S), 1)
    s = jnp.where(rows >= cols, s, jnp.float32(-1e9))
    m = jnp.max(s, axis=-1, keepdims=True)
    p = jnp.exp(s - m)
    l = jnp.sum(p, axis=-1, keepdims=True)
    o = jnp.dot(p, v_ref[0], preferred_element_type=jnp.float32)
    o_ref[0] = o / l


def _attention(q, k, v, block_q=256):
    return pl.pallas_call(
        _attn_body,
        grid=(H, S // block_q),
        in_specs=[
            pl.BlockSpec((1, block_q, HD), lambda h, i: (h, i, 0)),
            pl.BlockSpec((1, S, HD), lambda h, i: (h, 0, 0)),
            pl.BlockSpec((1, S, HD), lambda h, i: (h, 0, 0)),
        ],
        out_specs=pl.BlockSpec((1, block_q, HD), lambda h, i: (h, i, 0)),
        out_shape=jax.ShapeDtypeStruct((H, S, HD), jnp.float32),
        compiler_params=pltpu.CompilerParams(
            dimension_semantics=("parallel", "parallel")),
    )(q, k, v)


def _proj_res_body(o_ref, w_ref, x_ref, y_ref):
    y_ref[...] = x_ref[...] + jnp.dot(o_ref[...], w_ref[...],
                                      preferred_element_type=jnp.float32)


def _proj_residual(o, w, x, block_rows=256):
    return pl.pallas_call(
        _proj_res_body,
        grid=(T // block_rows,),
        in_specs=[
            pl.BlockSpec((block_rows, D), lambda i: (i, 0)),
            pl.BlockSpec((D, D), lambda i: (0, 0)),
            pl.BlockSpec((block_rows, D), lambda i: (i, 0)),
        ],
        out_specs=pl.BlockSpec((block_rows, D), lambda i: (i, 0)),
        out_shape=jax.ShapeDtypeStruct((T, D), jnp.float32),
        compiler_params=pltpu.CompilerParams(
            dimension_semantics=("parallel",)),
    )(o, w, x)


_HI_MASK = -65536


def _pack_bf16_pair(y):
    yb = lax.bitcast_convert_type(y.astype(jnp.bfloat16).astype(jnp.float32),
                                  jnp.int32)
    half = y.shape[1] // 2
    lo = lax.shift_right_logical(yb[:, :half], 16)
    hi = yb[:, half:]
    return hi | lo


def _unpack_bf16_pair(p):
    lo = lax.bitcast_convert_type(lax.shift_left(p, 16), jnp.float32)
    hi = lax.bitcast_convert_type(p & _HI_MASK, jnp.float32)
    return jnp.concatenate([lo, hi], axis=1)


def _norm2_body(x_ref, nw_ref, rw_ref, f_ref, l_ref):
    x = x_ref[...]
    h = x * lax.rsqrt(jnp.mean(x * x, axis=-1, keepdims=True) + EPS) * nw_ref[...]
    f_ref[...] = _pack_bf16_pair(h)
    l_ref[...] = jnp.dot(h, rw_ref[...], preferred_element_type=jnp.float32)


def _norm_and_router(x1, nw, rw_pad, block_rows=256):
    return pl.pallas_call(
        _norm2_body,
        grid=(T // block_rows,),
        in_specs=[
            pl.BlockSpec((block_rows, D), lambda i: (i, 0)),
            pl.BlockSpec((1, D), lambda i: (0, 0)),
            pl.BlockSpec((D, 128), lambda i: (0, 0)),
        ],
        out_specs=[
            pl.BlockSpec((block_rows, D // 2), lambda i: (i, 0)),
            pl.BlockSpec((block_rows, 128), lambda i: (i, 0)),
        ],
        out_shape=[
            jax.ShapeDtypeStruct((T, D // 2), jnp.int32),
            jax.ShapeDtypeStruct((T, 128), jnp.float32),
        ],
        compiler_params=pltpu.CompilerParams(
            dimension_semantics=("parallel",)),
    )(x1, nw, rw_pad)


def _ffn_body(te_ref, xg_ref, w1_ref, b1_ref, w2_ref, b2_ref, o_ref):
    x = _unpack_bf16_pair(xg_ref[...]).astype(jnp.bfloat16)
    h = jnp.dot(x, w1_ref[0], preferred_element_type=jnp.float32) + b1_ref[0]
    h = jax.nn.gelu(h)
    y = jnp.dot(h.astype(jnp.bfloat16), w2_ref[0],
                preferred_element_type=jnp.float32) + b2_ref[0]
    o_ref[...] = _pack_bf16_pair(y)


def _moe_ffn(xg, tile_expert, w1, b1, w2, b2):
    grid_spec = pltpu.PrefetchScalarGridSpec(
        num_scalar_prefetch=1,
        grid=(NUM_TILES,),
        in_specs=[
            pl.BlockSpec((TILE, D // 2), lambda t, te: (t, 0)),
            pl.BlockSpec((1, D, FF), lambda t, te: (te[t], 0, 0)),
            pl.BlockSpec((1, 1, FF), lambda t, te: (te[t], 0, 0)),
            pl.BlockSpec((1, FF, D), lambda t, te: (te[t], 0, 0)),
            pl.BlockSpec((1, 1, D), lambda t, te: (te[t], 0, 0)),
        ],
        out_specs=pl.BlockSpec((TILE, D // 2), lambda t, te: (t, 0)),
    )
    return pl.pallas_call(
        _ffn_body,
        grid_spec=grid_spec,
        out_shape=jax.ShapeDtypeStruct((PAD_T, D // 2), jnp.int32),
    )(tile_expert, xg, w1, b1, w2, b2)


def _combine_body(x1_ref, a0_ref, a1_ref, s0_ref, s1_ref, y_ref):
    a0 = _unpack_bf16_pair(a0_ref[...])
    a1 = _unpack_bf16_pair(a1_ref[...])
    y_ref[...] = (x1_ref[...] + a0 * s0_ref[...] + a1 * s1_ref[...])


def _combine(x1, a0, a1, s0, s1, block_rows=512):
    return pl.pallas_call(
        _combine_body,
        grid=(T // block_rows,),
        in_specs=[
            pl.BlockSpec((block_rows, D), lambda i: (i, 0)),
            pl.BlockSpec((block_rows, D // 2), lambda i: (i, 0)),
            pl.BlockSpec((block_rows, D // 2), lambda i: (i, 0)),
            pl.BlockSpec((block_rows, 1), lambda i: (i, 0)),
            pl.BlockSpec((block_rows, 1), lambda i: (i, 0)),
        ],
        out_specs=pl.BlockSpec((block_rows, D), lambda i: (i, 0)),
        out_shape=jax.ShapeDtypeStruct((T, D), jnp.float32),
        compiler_params=pltpu.CompilerParams(
            dimension_semantics=("parallel",)),
    )(x1, a0, a1, s0, s1)


def _sc_gather_rows(table, idx, n_rows):
    w32 = table.shape[1]
    b_per_w = n_rows // _NW
    ch = b_per_w
    n_chunks = 1
    while ch > 128:
        ch //= 2
        n_chunks *= 2
    mesh = plsc.VectorSubcoreMesh(core_axis_name="c", subcore_axis_name="s",
                                  num_cores=_NC, num_subcores=_NS)

    @functools.partial(
        pl.kernel, mesh=mesh,
        out_type=jax.ShapeDtypeStruct((n_rows, w32), jnp.int32),
        scratch_types=[
            pltpu.VMEM((ch,), jnp.int32),
            pltpu.VMEM((ch, w32), jnp.int32),
            pltpu.SemaphoreType.DMA,
        ],
    )
    def k(table_hbm, idx_hbm, out_hbm, idx_v, rows_v, sem):
        wid = lax.axis_index("s") * _NC + lax.axis_index("c")
        base = wid * b_per_w
        for c in range(n_chunks):
            off = base + c * ch
            pltpu.sync_copy(idx_hbm.at[pl.ds(off, ch)], idx_v)
            pltpu.async_copy(table_hbm.at[idx_v], rows_v, sem).wait()
            pltpu.sync_copy(rows_v, out_hbm.at[pl.ds(off, ch)])

    return k(table, idx)


def kernel(x, attn_norm_w, Wq, Wk, Wv, Wo, mlp_norm_w, router_w, router_b,
           ew1, eb1, ew2, eb2):
    x2d = x.reshape(T, D)
    nw1 = attn_norm_w.reshape(1, D)
    nw2 = mlp_norm_w.reshape(1, D)

    wqkv = jnp.concatenate([Wq, Wk, Wv], axis=1)
    qkv = _norm_matmul(x2d, nw1, wqkv)
    qkv3 = qkv.reshape(S, 3, H, HD).transpose(1, 2, 0, 3)
    o = _attention(qkv3[0], qkv3[1], qkv3[2])
    o2d = o.transpose(1, 0, 2).reshape(T, D)
    x1 = _proj_residual(o2d, Wo, x2d)

    rw_pad = jnp.zeros((D, 128), jnp.float32).at[:, :E].set(router_w)
    flat, logits_pad = _norm_and_router(x1, nw2, rw_pad)
    logits = logits_pad[:, :E] + router_b
    probs = jax.nn.softmax(logits, axis=-1)
    topk_scores, topk_idx = lax.top_k(probs, K)

    mean_probs = probs.mean(axis=0)
    flat_sel = topk_idx.reshape(-1)
    oh = (flat_sel[:, None] == jnp.arange(E)[None, :]).astype(jnp.int32)
    counts = oh.sum(axis=0)
    expert_counts = counts.astype(jnp.int32)
    fraction = counts.astype(jnp.float32) / jnp.float32(T * K)
    balancing_loss = jnp.float32(E) * jnp.sum(mean_probs * fraction)

    rank_r = (jnp.cumsum(oh, axis=0) * oh).sum(axis=1) - 1
    seg_pad = ((counts + TILE - 1) // TILE) * TILE
    po = jnp.concatenate([jnp.zeros((1,), jnp.int32),
                          jnp.cumsum(seg_pad)]).astype(jnp.int32)
    p_of_r = po[flat_sel] + rank_r
    valid_r = (rank_r < CAP).astype(jnp.float32)
    sorted_src = jnp.zeros((PAD_T,), jnp.int32).at[p_of_r].set(
        jnp.arange(T * K, dtype=jnp.int32))
    gather_idx = sorted_src // K
    tile_expert = jnp.minimum(
        jnp.searchsorted(po[1:], jnp.arange(NUM_TILES, dtype=jnp.int32) * TILE,
                         side="right"), E - 1).astype(jnp.int32)

    w_scr = jnp.transpose(topk_scores).reshape(-1)
    wv = w_scr * valid_r
    s0 = wv[:T].reshape(T, 1)
    s1 = wv[T:].reshape(T, 1)

    xg = _sc_gather_rows(flat, gather_idx, PAD_T)

    out_sorted = _moe_ffn(xg, tile_expert,
                          ew1.astype(jnp.bfloat16), eb1.reshape(E, 1, FF),
                          ew2.astype(jnp.bfloat16), eb2.reshape(E, 1, D))

    ab = _sc_gather_rows(out_sorted, p_of_r.astype(jnp.int32), 2 * T)
    a0, a1 = ab[:T], ab[T:]

    x2 = _combine(x1, a0, a1, s0, s1)
    return (x2.reshape(B, S, D), balancing_loss, expert_counts)
